# baseline (speedup 1.0000x reference)
import functools

import jax
import jax.numpy as jnp
from jax import lax
from jax.experimental import pallas as pl
from jax.experimental.pallas import tpu as pltpu
from jax.experimental.pallas import tpu_sc as plsc

NC = 2
NS = 16
NW = NC * NS
L = 16

B = 32
C = 5
HW = 256 * 256
ROW = C * HW
CHUNK = 16384
NCHUNK = HW // CHUNK

_LN2 = 0.6931471805599453
_SQRT2 = 1.4142135381698608


_LOG_COEF = (-0.09784370581217113, 0.20169463387552755, -0.25776693966811365,
             0.3340358402576027, -0.4997817521976763, 0.9999845821702478,
             -1.0281803654171862e-06 - 127.0 * _LN2)


def _vlog(x):
    bits = lax.bitcast_convert_type(x, jnp.int32)
    eb = jnp.right_shift(bits + 0x00400000, 23)
    m = lax.bitcast_convert_type(
        bits - jnp.left_shift(eb, 23) + (127 << 23), jnp.float32)
    t = m - 1.0
    p = _LOG_COEF[0]
    for c in _LOG_COEF[1:]:
        p = p * t + c
    return eb.astype(jnp.float32) * _LN2 + p


_mesh = plsc.VectorSubcoreMesh(core_axis_name="c", subcore_axis_name="s")


UNROLL = 4


CROWS = 64


@functools.partial(
    pl.kernel,
    out_type=jax.ShapeDtypeStruct((NW, 4, L), jnp.float32),
    mesh=_mesh,
    compiler_params=pltpu.CompilerParams(needs_layout_passes=False),
    scratch_types=[
        pltpu.VMEM((CROWS, 256), jnp.float32),
        pltpu.VMEM((CROWS, 256), jnp.float32),
        pltpu.VMEM((CROWS, 256), jnp.float32),
        pltpu.VMEM((CROWS, 256), jnp.float32),
        pltpu.VMEM((4, L), jnp.float32),
        pltpu.SemaphoreType.DMA,
    ],
)
def _partials(o_hbm, t_hbm, out_hbm, o_v0, t_v0, o_v1, t_v1, acc_v, sem):
    wid = lax.axis_index("s") * NC + lax.axis_index("c")
    bufs = ((o_v0, t_v0), (o_v1, t_v1))

    def start(c, ob, tb):
        h1 = pltpu.async_copy(
            o_hbm.at[wid, 4, pl.ds(c * CROWS, CROWS), :], ob, sem)
        h2 = pltpu.async_copy(
            t_hbm.at[wid, 4, pl.ds(c * CROWS, CROWS), :], tb, sem)
        return h1, h2

    def compute_chunk(ob, tb, carry):
        def body(i, carry):
            accs = list(carry)
            r = jnp.right_shift(i, 2)
            base = jnp.bitwise_and(i, 3) * (L * UNROLL)
            for u in range(UNROLL):
                o = ob[r, pl.ds(base + u * L, L)]
                y = tb[r, pl.ds(base + u * L, L)]
                l = (1.0 - y) * o + _vlog(1.0 + jnp.exp(-o))
                negm = (y > 0.0) & (y < 0.8)
                posm = y >= 1.0
                g = 4 * (u % 2)
                accs[g + 0] = accs[g + 0] + jnp.where(negm, l, 0.0)
                accs[g + 1] = accs[g + 1] + plsc.all_reduce_population_count(negm)
                accs[g + 2] = accs[g + 2] + jnp.where(posm, l, 0.0)
                accs[g + 3] = accs[g + 3] + plsc.all_reduce_population_count(posm)
            return tuple(accs)

        return plsc.parallel_loop(
            0, CROWS * 256 // (L * UNROLL), 1, unroll=2,
            carry=tuple(carry))(body)

    h = start(0, *bufs[0])
    zf = jnp.zeros((L,), jnp.float32)
    zi = jnp.zeros((L,), jnp.int32)
    carry = (zf, zi, zf, zi, zf, zi, zf, zi)
    for c in range(NCHUNK):
        h[0].wait()
        h[1].wait()
        if c + 1 < NCHUNK:
            nxt = start(c + 1, *bufs[(c + 1) % 2])
        carry = compute_chunk(*bufs[c % 2], carry)
        if c + 1 < NCHUNK:
            h = nxt
    acc_v[0, :] = carry[0] + carry[4]
    acc_v[1, :] = (carry[1] + carry[5]).astype(jnp.float32) * (1.0 / L)
    acc_v[2, :] = carry[2] + carry[6]
    acc_v[3, :] = (carry[3] + carry[7]).astype(jnp.float32) * (1.0 / L)
    pltpu.sync_copy(acc_v, out_hbm.at[wid])


@functools.partial(
    pl.kernel,
    out_type=jax.ShapeDtypeStruct((1,), jnp.float32),
    mesh=_mesh,
    compiler_params=pltpu.CompilerParams(needs_layout_passes=False),
    scratch_types=[
        pltpu.VMEM((NW, 4, L), jnp.float32),
        pltpu.VMEM((L,), jnp.float32),
    ],
)
def _combine(parts_hbm, out_hbm, parts_v, res_v):
    wid = lax.axis_index("s") * NC + lax.axis_index("c")

    @pl.when(wid == 0)
    def _():
        pltpu.sync_copy(parts_hbm, parts_v)

        def body(w, carry):
            ns, nc, ps, pc = carry
            return (ns + parts_v[w, 0, :], nc + parts_v[w, 1, :],
                    ps + parts_v[w, 2, :], pc + parts_v[w, 3, :])

        zero = jnp.zeros((L,), jnp.float32)
        ns, nc, ps, pc = lax.fori_loop(0, NW, body, (zero, zero, zero, zero))
        ns_v = jnp.full((L,), jnp.sum(ns), jnp.float32)
        nc_v = jnp.full((L,), jnp.sum(nc), jnp.float32)
        ps_v = jnp.full((L,), jnp.sum(ps), jnp.float32)
        pc_v = jnp.full((L,), jnp.sum(pc), jnp.float32)
        loss = ns_v / jnp.maximum(nc_v, 1.0) + ps_v / jnp.maximum(pc_v, 1.0)
        res_v[...] = jnp.where(pc_v > 0.0, loss, jnp.zeros((L,), jnp.float32))
        pltpu.sync_copy(res_v.at[pl.ds(0, 1)], out_hbm)


def kernel(output, target, branch, step):
    parts = _partials(output, target)
    return _combine(parts).reshape(())


# speedup vs baseline: 1.0865x; 1.0865x over previous
import functools

import jax
import jax.numpy as jnp
from jax import lax
from jax.experimental import pallas as pl
from jax.experimental.pallas import tpu as pltpu
from jax.experimental.pallas import tpu_sc as plsc

NC = 2
NS = 16
NW = NC * NS
L = 16

B = 32
C = 5
HW = 256 * 256
ROW = C * HW
CHUNK = 16384
NCHUNK = HW // CHUNK

_LN2 = 0.6931471805599453
_SQRT2 = 1.4142135381698608


_LOG_COEF = (-0.09784370581217113, 0.20169463387552755, -0.25776693966811365,
             0.3340358402576027, -0.4997817521976763, 0.9999845821702478,
             -1.0281803654171862e-06 - 127.0 * _LN2)


def _vlog(x):
    bits = lax.bitcast_convert_type(x, jnp.int32)
    eb = jnp.right_shift(bits + 0x00400000, 23)
    m = lax.bitcast_convert_type(
        bits - jnp.left_shift(eb, 23) + (127 << 23), jnp.float32)
    t = m - 1.0
    p = _LOG_COEF[0]
    for c in _LOG_COEF[1:]:
        p = p * t + c
    return eb.astype(jnp.float32) * _LN2 + p


_mesh = plsc.VectorSubcoreMesh(core_axis_name="c", subcore_axis_name="s")


UNROLL = 2


CROWS = 64


@functools.partial(
    pl.kernel,
    out_type=jax.ShapeDtypeStruct((NW, 4, L), jnp.float32),
    mesh=_mesh,
    compiler_params=pltpu.CompilerParams(needs_layout_passes=False),
    scratch_types=[
        pltpu.VMEM((CROWS, 256), jnp.float32),
        pltpu.VMEM((CROWS, 256), jnp.float32),
        pltpu.VMEM((CROWS, 256), jnp.float32),
        pltpu.VMEM((CROWS, 256), jnp.float32),
        pltpu.VMEM((4, L), jnp.float32),
        pltpu.SemaphoreType.DMA,
    ],
)
def _partials(o_hbm, t_hbm, out_hbm, o_v0, t_v0, o_v1, t_v1, acc_v, sem):
    wid = lax.axis_index("s") * NC + lax.axis_index("c")
    bufs = ((o_v0, t_v0), (o_v1, t_v1))

    def start(c, ob, tb):
        h1 = pltpu.async_copy(
            o_hbm.at[wid, 4, pl.ds(c * CROWS, CROWS), :], ob, sem)
        h2 = pltpu.async_copy(
            t_hbm.at[wid, 4, pl.ds(c * CROWS, CROWS), :], tb, sem)
        return h1, h2

    def compute_chunk(ob, tb, carry):
        def body(i, carry):
            accs = list(carry)
            r = jnp.right_shift(i, 2)
            base = jnp.bitwise_and(i, 3) * (L * UNROLL)
            for u in range(UNROLL):
                o = ob[r, pl.ds(base + u * L, L)]
                y = tb[r, pl.ds(base + u * L, L)]
                l = (1.0 - y) * o + _vlog(1.0 + jnp.exp(-o))
                negm = (y > 0.0) & (y < 0.8)
                posm = y >= 1.0
                g = 4 * (u % 2)
                accs[g + 0] = accs[g + 0] + jnp.where(negm, l, 0.0)
                accs[g + 1] = accs[g + 1] + plsc.all_reduce_population_count(negm)
                accs[g + 2] = accs[g + 2] + jnp.where(posm, l, 0.0)
                accs[g + 3] = accs[g + 3] + plsc.all_reduce_population_count(posm)
            return tuple(accs)

        return plsc.parallel_loop(
            0, CROWS * 256 // (L * UNROLL), 1, carry=tuple(carry))(body)

    h = start(0, *bufs[0])
    zf = jnp.zeros((L,), jnp.float32)
    zi = jnp.zeros((L,), jnp.int32)
    carry = (zf, zi, zf, zi, zf, zi, zf, zi)
    for c in range(NCHUNK):
        h[0].wait()
        h[1].wait()
        if c + 1 < NCHUNK:
            nxt = start(c + 1, *bufs[(c + 1) % 2])
        carry = compute_chunk(*bufs[c % 2], carry)
        if c + 1 < NCHUNK:
            h = nxt
    acc_v[0, :] = carry[0] + carry[4]
    acc_v[1, :] = (carry[1] + carry[5]).astype(jnp.float32) * (1.0 / L)
    acc_v[2, :] = carry[2] + carry[6]
    acc_v[3, :] = (carry[3] + carry[7]).astype(jnp.float32) * (1.0 / L)
    pltpu.sync_copy(acc_v, out_hbm.at[wid])


@functools.partial(
    pl.kernel,
    out_type=jax.ShapeDtypeStruct((1,), jnp.float32),
    mesh=_mesh,
    compiler_params=pltpu.CompilerParams(needs_layout_passes=False),
    scratch_types=[
        pltpu.VMEM((NW, 4, L), jnp.float32),
        pltpu.VMEM((L,), jnp.float32),
    ],
)
def _combine(parts_hbm, out_hbm, parts_v, res_v):
    wid = lax.axis_index("s") * NC + lax.axis_index("c")

    @pl.when(wid == 0)
    def _():
        pltpu.sync_copy(parts_hbm, parts_v)

        def body(w, carry):
            ns, nc, ps, pc = carry
            return (ns + parts_v[w, 0, :], nc + parts_v[w, 1, :],
                    ps + parts_v[w, 2, :], pc + parts_v[w, 3, :])

        zero = jnp.zeros((L,), jnp.float32)
        ns, nc, ps, pc = lax.fori_loop(0, NW, body, (zero, zero, zero, zero))
        ns_v = jnp.full((L,), jnp.sum(ns), jnp.float32)
        nc_v = jnp.full((L,), jnp.sum(nc), jnp.float32)
        ps_v = jnp.full((L,), jnp.sum(ps), jnp.float32)
        pc_v = jnp.full((L,), jnp.sum(pc), jnp.float32)
        loss = ns_v / jnp.maximum(nc_v, 1.0) + ps_v / jnp.maximum(pc_v, 1.0)
        res_v[...] = jnp.where(pc_v > 0.0, loss, jnp.zeros((L,), jnp.float32))
        pltpu.sync_copy(res_v.at[pl.ds(0, 1)], out_hbm)


def kernel(output, target, branch, step):
    parts = _partials(output, target)
    return _combine(parts).reshape(())


# speedup vs baseline: 1.2016x; 1.1060x over previous
import functools

import jax
import jax.numpy as jnp
from jax import lax
from jax.experimental import pallas as pl
from jax.experimental.pallas import tpu as pltpu
from jax.experimental.pallas import tpu_sc as plsc

NC = 2
NS = 16
NW = NC * NS
L = 16

B = 32
C = 5
HW = 256 * 256
ROW = C * HW
CHUNK = 16384
NCHUNK = HW // CHUNK

_LN2 = 0.6931471805599453
_SQRT2 = 1.4142135381698608


_LOG_COEF = (-0.17527881874048135, 1.04415463971721, -2.5870129281192304,
             3.742640708592355,
             -2.0244418109730313 - 127.0 * _LN2)


def _vlog(x):
    bits = lax.bitcast_convert_type(x, jnp.int32)
    eb = jnp.right_shift(bits + 0x00400000, 23)
    m = lax.bitcast_convert_type(
        bits - jnp.left_shift(eb, 23) + (127 << 23), jnp.float32)
    p = _LOG_COEF[0]
    for c in _LOG_COEF[1:]:
        p = p * m + c
    return eb.astype(jnp.float32) * _LN2 + p


_mesh = plsc.VectorSubcoreMesh(core_axis_name="c", subcore_axis_name="s")


UNROLL = 4
IPR = 256 // (L * UNROLL)
IPR_SHIFT = IPR.bit_length() - 1


CROWS = 64


@functools.partial(
    pl.kernel,
    out_type=jax.ShapeDtypeStruct((NW, 4, L), jnp.float32),
    mesh=_mesh,
    compiler_params=pltpu.CompilerParams(needs_layout_passes=False),
    scratch_types=[
        pltpu.VMEM((CROWS, 256), jnp.float32),
        pltpu.VMEM((CROWS, 256), jnp.float32),
        pltpu.VMEM((CROWS, 256), jnp.float32),
        pltpu.VMEM((CROWS, 256), jnp.float32),
        pltpu.VMEM((4, L), jnp.float32),
        pltpu.SemaphoreType.DMA,
    ],
)
def _partials(o_hbm, t_hbm, out_hbm, o_v0, t_v0, o_v1, t_v1, acc_v, sem):
    wid = lax.axis_index("s") * NC + lax.axis_index("c")
    bufs = ((o_v0, t_v0), (o_v1, t_v1))

    def start(c, ob, tb):
        h1 = pltpu.async_copy(
            o_hbm.at[wid, 4, pl.ds(c * CROWS, CROWS), :], ob, sem)
        h2 = pltpu.async_copy(
            t_hbm.at[wid, 4, pl.ds(c * CROWS, CROWS), :], tb, sem)
        return h1, h2

    def compute_chunk(ob, tb, carry):
        def body(i, carry):
            accs = list(carry)
            r = jnp.right_shift(i, IPR_SHIFT)
            base = jnp.bitwise_and(i, IPR - 1) * (L * UNROLL)
            for u in range(UNROLL):
                o = ob[r, pl.ds(base + u * L, L)]
                y = tb[r, pl.ds(base + u * L, L)]
                l = _vlog(1.0 + jnp.exp(o)) - y * o
                negm = (y > 0.0) & (y < 0.8)
                posm = y >= 1.0
                g = 4 * (u % 2)
                accs[g + 0] = accs[g + 0] + jnp.where(negm, l, 0.0)
                accs[g + 1] = accs[g + 1] + plsc.all_reduce_population_count(negm)
                accs[g + 2] = accs[g + 2] + jnp.where(posm, l, 0.0)
                accs[g + 3] = accs[g + 3] + plsc.all_reduce_population_count(posm)
            return tuple(accs)

        return plsc.parallel_loop(
            0, CROWS * 256 // (L * UNROLL), 1, carry=tuple(carry))(body)

    h = start(0, *bufs[0])
    zf = jnp.zeros((L,), jnp.float32)
    zi = jnp.zeros((L,), jnp.int32)
    carry = (zf, zi, zf, zi, zf, zi, zf, zi)
    for c in range(NCHUNK):
        h[0].wait()
        h[1].wait()
        if c + 1 < NCHUNK:
            nxt = start(c + 1, *bufs[(c + 1) % 2])
        carry = compute_chunk(*bufs[c % 2], carry)
        if c + 1 < NCHUNK:
            h = nxt
    acc_v[0, :] = carry[0] + carry[4]
    acc_v[1, :] = (carry[1] + carry[5]).astype(jnp.float32) * (1.0 / L)
    acc_v[2, :] = carry[2] + carry[6]
    acc_v[3, :] = (carry[3] + carry[7]).astype(jnp.float32) * (1.0 / L)
    pltpu.sync_copy(acc_v, out_hbm.at[wid])


@functools.partial(
    pl.kernel,
    out_type=jax.ShapeDtypeStruct((1,), jnp.float32),
    mesh=_mesh,
    compiler_params=pltpu.CompilerParams(needs_layout_passes=False),
    scratch_types=[
        pltpu.VMEM((NW, 4, L), jnp.float32),
        pltpu.VMEM((L,), jnp.float32),
    ],
)
def _combine(parts_hbm, out_hbm, parts_v, res_v):
    wid = lax.axis_index("s") * NC + lax.axis_index("c")

    @pl.when(wid == 0)
    def _():
        pltpu.sync_copy(parts_hbm, parts_v)

        def body(w, carry):
            ns, nc, ps, pc = carry
            return (ns + parts_v[w, 0, :], nc + parts_v[w, 1, :],
                    ps + parts_v[w, 2, :], pc + parts_v[w, 3, :])

        zero = jnp.zeros((L,), jnp.float32)
        ns, nc, ps, pc = lax.fori_loop(0, NW, body, (zero, zero, zero, zero))
        ns_v = jnp.full((L,), jnp.sum(ns), jnp.float32)
        nc_v = jnp.full((L,), jnp.sum(nc), jnp.float32)
        ps_v = jnp.full((L,), jnp.sum(ps), jnp.float32)
        pc_v = jnp.full((L,), jnp.sum(pc), jnp.float32)
        loss = ns_v / jnp.maximum(nc_v, 1.0) + ps_v / jnp.maximum(pc_v, 1.0)
        res_v[...] = jnp.where(pc_v > 0.0, loss, jnp.zeros((L,), jnp.float32))
        pltpu.sync_copy(res_v.at[pl.ds(0, 1)], out_hbm)


def kernel(output, target, branch, step):
    parts = _partials(output, target)
    return _combine(parts).reshape(())
